# Initial kernel scaffold; baseline (speedup 1.0000x reference)
import jax
import jax.numpy as jnp
from jax.experimental import pallas as pl


def kernel(h, edge_index, edge_weight, edge_attr, data, W0, b0, Wf, bf, Ws, bs, gamma, beta):
    raise NotImplementedError("write your pallas kernel here")


# speedup vs baseline: 1.3455x; 1.3455x over previous
import functools

import jax
import jax.numpy as jnp
from jax import lax
from jax.experimental import pallas as pl
from jax.experimental.pallas import tpu as pltpu
from jax.experimental.pallas import tpu_sc as plsc

EPS = 1e-5


def _softplus(x):
    return jnp.maximum(x, 0.0) + jnp.log1p(jnp.exp(-jnp.abs(x)))


def _node_proj(h, W0, b0, Wi, Wj):
    N, DIN = h.shape
    C = W0.shape[1]
    Z = Wi.shape[1]
    BN_ = 2000
    assert N % BN_ == 0

    def body(h_ref, w_ref, b_ref, wi_ref, wj_ref, o_ref, ti_ref, tj_ref):
        x = jnp.dot(h_ref[...], w_ref[...], preferred_element_type=jnp.float32)
        o = _softplus(x + b_ref[...])
        o_ref[...] = o
        ti_ref[...] = jnp.dot(o, wi_ref[...], preferred_element_type=jnp.float32)
        tj_ref[...] = jnp.dot(o, wj_ref[...], preferred_element_type=jnp.float32)

    return pl.pallas_call(
        body,
        grid=(N // BN_,),
        in_specs=[
            pl.BlockSpec((BN_, DIN), lambda i: (i, 0)),
            pl.BlockSpec((DIN, C), lambda i: (0, 0)),
            pl.BlockSpec((1, C), lambda i: (0, 0)),
            pl.BlockSpec((C, Z), lambda i: (0, 0)),
            pl.BlockSpec((C, Z), lambda i: (0, 0)),
        ],
        out_specs=[
            pl.BlockSpec((BN_, C), lambda i: (i, 0)),
            pl.BlockSpec((BN_, Z), lambda i: (i, 0)),
            pl.BlockSpec((BN_, Z), lambda i: (i, 0)),
        ],
        out_shape=[
            jax.ShapeDtypeStruct((N, C), jnp.float32),
            jax.ShapeDtypeStruct((N, Z), jnp.float32),
            jax.ShapeDtypeStruct((N, Z), jnp.float32),
        ],
    )(h, W0, b0, Wi, Wj)


def _edge_gather(ti, tj, src, dst):
    N, Z = ti.shape
    E = src.shape[0]
    NC, NS = 2, 16
    NW = NC * NS
    assert E % NW == 0
    EPW = E // NW
    K = 256
    NFULL = EPW // K
    REM = EPW - NFULL * K
    RG, RT = REM // 128, REM % 128
    NL = Z // 16

    mesh = plsc.VectorSubcoreMesh(core_axis_name="c", subcore_axis_name="s")

    @functools.partial(
        pl.kernel,
        mesh=mesh,
        out_type=jax.ShapeDtypeStruct((E, Z), jnp.float32),
        scratch_types=[
            pltpu.VMEM((2, 128), jnp.int32),
            pltpu.VMEM((2, 128), jnp.int32),
            pltpu.VMEM((K, Z), jnp.float32),
            pltpu.VMEM((K, Z), jnp.float32),
            pltpu.SemaphoreType.DMA,
        ],
    )
    def _k(ti_hbm, tj_hbm, src_hbm, dst_hbm, g_hbm, di, si, xi_v, xj_v, sem):
        wid = lax.axis_index("s") * NC + lax.axis_index("c")
        base0 = wid * EPW

        def do_group(base, j, n):
            pltpu.sync_copy(dst_hbm.at[pl.ds(base + j * 128, n)],
                            di.at[j, pl.ds(0, n)])
            pltpu.sync_copy(src_hbm.at[pl.ds(base + j * 128, n)],
                            si.at[j, pl.ds(0, n)])
            a = pltpu.async_copy(ti_hbm.at[di.at[j, pl.ds(0, n)]],
                                 xi_v.at[pl.ds(j * 128, n)], sem)
            b = pltpu.async_copy(tj_hbm.at[si.at[j, pl.ds(0, n)]],
                                 xj_v.at[pl.ds(j * 128, n)], sem)
            a.wait()
            b.wait()

        def add_rows(nrows):
            def arow(r, carry):
                for col in range(NL):
                    s = pl.ds(col * 16, 16)
                    xi_v[r, s] = xi_v[r, s] + xj_v[r, s]
                return carry
            lax.fori_loop(0, nrows, arow, 0)

        def body(i, carry):
            base = base0 + i * K
            for j in range(K // 128):
                do_group(base, j, 128)
            add_rows(K)
            pltpu.sync_copy(xi_v, g_hbm.at[pl.ds(base, K)])
            return carry

        lax.fori_loop(0, NFULL, body, 0)

        if REM:
            base = base0 + NFULL * K
            for j in range(RG):
                do_group(base, j, 128)
            if RT:
                do_group(base, RG, RT)
            add_rows(REM)
            pltpu.sync_copy(xi_v.at[pl.ds(0, REM)], g_hbm.at[pl.ds(base, REM)])

    return _k(ti, tj, src, dst)


def _edge_mlp(g, ea, We, bb):
    E, Z = g.shape
    C = Z // 2
    DE = ea.shape[1]
    H = C // 4
    BE = 2000
    assert E % BE == 0

    def body(g_ref, ea_ref, we_ref, bb_ref, m_ref):
        acc = g_ref[...] + jnp.dot(ea_ref[...], we_ref[...],
                                   preferred_element_type=jnp.float32)
        acc = acc + bb_ref[...]
        f = acc[:, :C]
        s = acc[:, C:]
        msg = (1.0 / (1.0 + jnp.exp(-f))) * _softplus(s)
        for q in range(4):
            m_ref[q] = msg[:, q * H:(q + 1) * H]

    return pl.pallas_call(
        body,
        grid=(E // BE,),
        in_specs=[
            pl.BlockSpec((BE, Z), lambda i: (i, 0)),
            pl.BlockSpec((BE, DE), lambda i: (i, 0)),
            pl.BlockSpec((DE, Z), lambda i: (0, 0)),
            pl.BlockSpec((1, Z), lambda i: (0, 0)),
        ],
        out_specs=pl.BlockSpec((4, BE, H), lambda i: (0, i, 0)),
        out_shape=jax.ShapeDtypeStruct((4, E, H), jnp.float32),
    )(g, ea, We, bb)


def _scatter_quarters(m_flat, dst, N):
    E = dst.shape[0]
    H = m_flat.shape[1]
    NC, NS = 2, 16
    assert E % NS == 0
    EPT = E // NS
    K = 512
    NFULL = EPT // K
    REM = EPT - NFULL * K
    RG, RT = REM // 128, REM % 128
    STRIPE = 3128
    NPAD = NS * STRIPE
    assert NPAD >= N and (NS - 1) * STRIPE < N
    LAST = N - (NS - 1) * STRIPE
    ZFULL = STRIPE // K
    ZREM = STRIPE - ZFULL * K

    mesh = plsc.VectorSubcoreMesh(core_axis_name="c", subcore_axis_name="s")

    @functools.partial(
        pl.kernel,
        mesh=mesh,
        out_type=jax.ShapeDtypeStruct((4 * NPAD, H), jnp.float32),
        scratch_types=[
            pltpu.VMEM((128,), jnp.int32),
            pltpu.VMEM((128,), jnp.int32),
            pltpu.VMEM((128,), jnp.int32),
            pltpu.VMEM((128,), jnp.int32),
            pltpu.VMEM((128,), jnp.int32),
            pltpu.VMEM((128, H), jnp.float32),
            pltpu.VMEM((128, H), jnp.float32),
            pltpu.VMEM((128, H), jnp.float32),
            pltpu.VMEM((128, H), jnp.float32),
            pltpu.VMEM_SHARED((NPAD, H), jnp.float32),
            pltpu.SemaphoreType.DMA,
        ],
    )
    def _k(m_hbm, dst_hbm, a_hbm, di0, di1, di2, di3, dz,
           u0, u1, u2, u3, acc, sem):
        dis = [di0, di1, di2, di3]
        us = [u0, u1, u2, u3]
        c = lax.axis_index("c")
        sid = lax.axis_index("s")

        zv = jnp.zeros((16,), jnp.float32)
        r0 = sid * STRIPE
        ZB = STRIPE // 128
        ZT = STRIPE - ZB * 128

        def set_iota_row(base):
            for t in range(8):
                dz[pl.ds(t * 16, 16)] = (
                    lax.iota(jnp.int32, 16) + (base + t * 16))

        def zero_acc():
            def zrow(i, carry):
                u0[i, pl.ds(0, 16)] = zv
                return carry

            lax.fori_loop(0, 128, zrow, 0)
            for kb in range(ZB):
                set_iota_row(r0 + kb * 128)
                pltpu.sync_copy(u0, acc.at[dz])
            io = lax.iota(jnp.int32, 16)
            for t in range(3):
                dz[pl.ds(t * 16, 16)] = io + (r0 + ZB * 128 + t * 16)
            dz[pl.ds(48, 16)] = jnp.where(
                io < 8, io + (r0 + ZB * 128 + 48), io - 8 + r0)
            for t in range(4, 8):
                dz[pl.ds(t * 16, 16)] = io + (r0 + 8 + (t - 4) * 16)
            pltpu.sync_copy(u0, acc.at[dz])

        def do_scatter(q):
            base0 = sid * EPT

            def group(base, j):
                pltpu.sync_copy(dst_hbm.at[pl.ds(base + j * 128, 128)],
                                dis[j])
                pltpu.sync_copy(m_hbm.at[pl.ds(q * E + base + j * 128, 128)],
                                us[j])
                pltpu.sync_copy(us[j], acc.at[dis[j]], add=True)

            def tail_group(base, j, n):
                pltpu.sync_copy(dst_hbm.at[pl.ds(base + j * 128, n)],
                                dis[j].at[pl.ds(0, n)])
                for t in range((128 - n) // 16):
                    dis[j][pl.ds(n + t * 16, 16)] = (
                        lax.iota(jnp.int32, 16) + t * 16)
                pltpu.sync_copy(m_hbm.at[pl.ds(q * E + base + j * 128, n)],
                                us[j].at[pl.ds(0, n)])

                def zfill(r, carry):
                    us[j][r, pl.ds(0, 16)] = zv
                    return carry

                lax.fori_loop(n, 128, zfill, 0)
                pltpu.sync_copy(us[j], acc.at[dis[j]], add=True)

            def body(i, carry):
                base = base0 + i * K
                for j in range(K // 128):
                    group(base, j)
                return carry

            lax.fori_loop(0, NFULL, body, 0)

            if REM:
                base = base0 + NFULL * K
                for j in range(RG):
                    group(base, j)
                if RT:
                    tail_group(base, RG, RT)

        for rnd in range(2):
            q = c * 2 + rnd
            zero_acc()
            plsc.subcore_barrier()
            do_scatter(q)
            plsc.subcore_barrier()
            for kb in range(ZB):
                set_iota_row(r0 + kb * 128)
                pltpu.sync_copy(acc.at[dz], u0)
                pltpu.sync_copy(
                    u0, a_hbm.at[pl.ds(q * NPAD + r0 + kb * 128, 128)])
            if ZT:
                set_iota_row(r0 + ZB * 128 - (128 - ZT))
                pltpu.sync_copy(acc.at[dz], u0)
                pltpu.sync_copy(
                    u0.at[pl.ds(128 - ZT, ZT)],
                    a_hbm.at[pl.ds(q * NPAD + r0 + ZB * 128, ZT)])

    return _k(m_flat, dst)


def _bn_finish(agg, out, gamma, beta):
    N, C = out.shape
    B = 2000
    assert N % B == 0
    inv_n = 1.0 / N

    def body(a_ref, o_ref, g_ref, b_ref, res_ref, acc_ref):
        p = pl.program_id(0)
        i = pl.program_id(1)

        @pl.when((p == 0) & (i == 0))
        def _():
            acc_ref[...] = jnp.zeros_like(acc_ref)

        agg = a_ref[...]

        @pl.when(p == 0)
        def _():
            acc_ref[0, :] = acc_ref[0, :] + jnp.sum(agg, axis=0)
            acc_ref[1, :] = acc_ref[1, :] + jnp.sum(agg * agg, axis=0)
            res_ref[...] = jnp.zeros_like(res_ref)

        @pl.when(p == 1)
        def _():
            mean = acc_ref[0, :] * inv_n
            var = acc_ref[1, :] * inv_n - mean * mean
            rstd = lax.rsqrt(var + EPS)
            normed = (agg - mean[None, :]) * (rstd[None, :] * g_ref[...]) + b_ref[...]
            res_ref[...] = normed + 2.0 * o_ref[...]

    return pl.pallas_call(
        body,
        grid=(2, N // B),
        in_specs=[
            pl.BlockSpec((B, C), lambda p, i: (i, 0)),
            pl.BlockSpec((B, C), lambda p, i: (i, 0)),
            pl.BlockSpec((1, C), lambda p, i: (0, 0)),
            pl.BlockSpec((1, C), lambda p, i: (0, 0)),
        ],
        out_specs=pl.BlockSpec((B, C), lambda p, i: (i, 0)),
        out_shape=jax.ShapeDtypeStruct((N, C), jnp.float32),
        scratch_shapes=[pltpu.VMEM((2, C), jnp.float32)],
    )(agg, out, gamma, beta)


def kernel(h, edge_index, edge_weight, edge_attr, data, W0, b0, Wf, bf, Ws, bs,
           gamma, beta):
    N = h.shape[0]
    C = W0.shape[1]
    src = edge_index[0]
    dst = edge_index[1]
    Wi = jnp.concatenate([Wf[:C], Ws[:C]], axis=1)
    Wj = jnp.concatenate([Wf[C:2 * C], Ws[C:2 * C]], axis=1)
    We = jnp.concatenate([Wf[2 * C:], Ws[2 * C:]], axis=1)
    bb = jnp.concatenate([bf, bs]).reshape(1, 2 * C)
    out, ti, tj = _node_proj(h, W0, b0.reshape(1, C), Wi, Wj)
    g = _edge_gather(ti, tj, src, dst)
    m_cat = _edge_mlp(g, edge_attr, We, bb)
    msg = jnp.concatenate([m_cat[0], m_cat[1], m_cat[2], m_cat[3]], axis=1)
    agg = jax.ops.segment_sum(msg, dst, num_segments=N)
    return _bn_finish(agg, out, gamma.reshape(1, C), beta.reshape(1, C))
